# Initial kernel scaffold; baseline (speedup 1.0000x reference)
import jax
import jax.numpy as jnp
from jax.experimental import pallas as pl


def kernel(x, parameter):
    raise NotImplementedError("write your pallas kernel here")


# speedup vs baseline: 4.3557x; 4.3557x over previous
import functools

import jax
import jax.numpy as jnp
from jax import lax
from jax.experimental import pallas as pl
from jax.experimental.pallas import tpu as pltpu
from jax.experimental.pallas import tpu_sc as plsc

_NC = 2
_NS = 16
_NW = _NC * _NS
_G = 128
_KB = 8
_ROWS_BUF = _G * _KB


@functools.lru_cache(maxsize=None)
def _make_gather(B, V, D):
    b_per_w = B // _NW
    n_outer = b_per_w // _ROWS_BUF
    mesh = plsc.VectorSubcoreMesh(core_axis_name="c", subcore_axis_name="s")

    @functools.partial(
        pl.kernel,
        mesh=mesh,
        compiler_params=pltpu.CompilerParams(use_tc_tiling_on_sc=False),
        out_type=jax.ShapeDtypeStruct((B, D), jnp.float32),
        scratch_types=[
            pltpu.VMEM((b_per_w,), jnp.int32),
            pltpu.VMEM((_ROWS_BUF, D), jnp.float32),
            pltpu.SemaphoreType.DMA,
        ],
    )
    def gather_kernel(idx_hbm, table_hbm, out_hbm, idx_v, rows_v, sem):
        wid = lax.axis_index("s") * _NC + lax.axis_index("c")
        base = wid * b_per_w
        pltpu.sync_copy(idx_hbm.at[pl.ds(base, b_per_w)], idx_v)

        def outer(i, carry):
            row0 = i * _ROWS_BUF
            copies = []
            for j in range(_KB):
                copies.append(pltpu.async_copy(
                    table_hbm.at[idx_v.at[pl.ds(row0 + j * _G, _G)]],
                    rows_v.at[pl.ds(j * _G, _G)],
                    sem,
                ))
            for c in copies:
                c.wait()
            pltpu.sync_copy(rows_v, out_hbm.at[pl.ds(base + row0, _ROWS_BUF)])
            return carry

        lax.fori_loop(0, n_outer, outer, 0)

    return gather_kernel


def kernel(x, parameter):
    L1, L2, orbit_num = x.shape[-3], x.shape[-2], x.shape[-1]
    lead = x.shape[:-3]
    flat = x.reshape(-1)
    B = flat.shape[0]
    V, D = parameter.shape

    result = _make_gather(B, V, D)(flat, parameter)
    result = result.reshape(lead + (L1 * L2 * orbit_num, D))

    p1 = jnp.broadcast_to(jnp.arange(L1).reshape(L1, 1, 1), (L1, L2, orbit_num))
    p2 = jnp.broadcast_to(jnp.arange(L2).reshape(1, L2, 1), (L1, L2, orbit_num))
    p3 = jnp.broadcast_to(jnp.arange(orbit_num).reshape(1, 1, orbit_num),
                          (L1, L2, orbit_num))
    position = jnp.stack([p1, p2, p3]).reshape(3, L1 * L2 * orbit_num)
    position = position.transpose(1, 0).astype(result.dtype)
    return (result, position)


# speedup vs baseline: 4.3854x; 1.0068x over previous
import functools

import jax
import jax.numpy as jnp
from jax import lax
from jax.experimental import pallas as pl
from jax.experimental.pallas import tpu as pltpu
from jax.experimental.pallas import tpu_sc as plsc

_NC = 2
_NS = 16
_NW = _NC * _NS
_G = 128
_KB = 8
_NI = _G * _KB
_PPW = 32
_PW = 4 * 8 * 8 * 128


@functools.lru_cache(maxsize=None)
def _make_gather(V, D):
    NB = _NW * _PPW
    mesh = plsc.VectorSubcoreMesh(core_axis_name="c", subcore_axis_name="s")

    @functools.partial(
        pl.kernel,
        mesh=mesh,
        compiler_params=pltpu.CompilerParams(use_tc_tiling_on_sc=False,
                                             needs_layout_passes=False),
        out_type=jax.ShapeDtypeStruct((NB * _PW,), jnp.float32),
        scratch_types=[
            pltpu.VMEM((16, 8, 4, 32), jnp.int32),
            pltpu.VMEM((_PPW * _NI,), jnp.int32),
            pltpu.VMEM((_NI, D), jnp.float32),
            pltpu.VMEM((_PW,), jnp.float32),
            pltpu.SemaphoreType.DMA,
        ],
    )
    def gather_kernel(idx_hbm, table_hbm, out_hbm, staged_v, ordered_v,
                      rows_v, out_v, sem):
        wid = lax.axis_index("s") * _NC + lax.axis_index("c")
        bt = wid // 4
        bl0 = (wid % 4) * 32
        iot = jnp.arange(16, dtype=jnp.int32)

        for half in range(2):
            pltpu.sync_copy(
                idx_hbm.at[:, pl.ds(half * 8, 8), bt, :, pl.ds(bl0, 32)],
                staged_v)

            def detr(g, carry):
                l1 = g // 8
                l2h = g % 8
                i0 = (l1 * 16 + half * 8 + l2h) * 4
                for o in range(4):
                    dbase = iot * 1024 + (i0 + o)
                    lo = staged_v[l1, l2h, o, pl.ds(0, 16)]
                    plsc.store_scatter(ordered_v, [dbase], lo)
                    hi = staged_v[l1, l2h, o, pl.ds(16, 16)]
                    plsc.store_scatter(ordered_v, [dbase + 16 * 1024], hi)
                return carry

            lax.fori_loop(0, 128, detr, 0)

        cvec = (iot // 8) * 8192 + (iot % 8) * 128

        def plane(p, carry):
            row0 = p * _NI
            copies = []
            for j in range(_KB):
                copies.append(pltpu.async_copy(
                    table_hbm.at[ordered_v.at[pl.ds(row0 + j * _G, _G)]],
                    rows_v.at[pl.ds(j * _G, _G)],
                    sem,
                ))
            for c in copies:
                c.wait()

            def transp(i, tcarry):
                base = (i // 128) * 1024 + i % 128
                lo = rows_v[i, pl.ds(0, 16)]
                plsc.store_scatter(out_v, [cvec + base], lo)
                hi = rows_v[i, pl.ds(16, 16)]
                plsc.store_scatter(out_v, [cvec + (base + 16384)], hi)
                return tcarry

            lax.fori_loop(0, _NI, transp, 0)
            pltpu.sync_copy(out_v,
                            out_hbm.at[pl.ds((wid * _PPW + p) * _PW, _PW)])
            return carry

        lax.fori_loop(0, _PPW, plane, 0)

    return gather_kernel


def kernel(x, parameter):
    L1, L2, orbit_num = x.shape[-3], x.shape[-2], x.shape[-1]
    lead = x.shape[:-3]
    NI = L1 * L2 * orbit_num
    V, D = parameter.shape

    x_phys = (x.transpose(1, 2, 3, 0)
               .reshape(L1, L2, orbit_num, 8, 128)
               .transpose(0, 1, 3, 2, 4))

    out1d = _make_gather(V, D)(x_phys, parameter)
    out5 = out1d.reshape(lead[0], 4, 8, 8, 128)
    result = out5.transpose(0, 2, 4, 1, 3).reshape(lead + (NI, D))

    p1 = jnp.broadcast_to(jnp.arange(L1).reshape(L1, 1, 1), (L1, L2, orbit_num))
    p2 = jnp.broadcast_to(jnp.arange(L2).reshape(1, L2, 1), (L1, L2, orbit_num))
    p3 = jnp.broadcast_to(jnp.arange(orbit_num).reshape(1, 1, orbit_num),
                          (L1, L2, orbit_num))
    position = jnp.stack([p1, p2, p3]).reshape(3, NI)
    position = position.transpose(1, 0).astype(result.dtype)
    return (result, position)


# speedup vs baseline: 5.6812x; 1.2955x over previous
import functools

import jax
import jax.numpy as jnp
from jax import lax
from jax.experimental import pallas as pl
from jax.experimental.pallas import tpu as pltpu
from jax.experimental.pallas import tpu_sc as plsc

_NC = 2
_NS = 16
_NW = _NC * _NS
_G = 128
_KB = 8
_NI = _G * _KB
_PPW = 32
_PW = 4 * 8 * 8 * 128


@functools.lru_cache(maxsize=None)
def _make_gather(V, D):
    NB = _NW * _PPW
    mesh = plsc.VectorSubcoreMesh(core_axis_name="c", subcore_axis_name="s")

    @functools.partial(
        pl.kernel,
        mesh=mesh,
        compiler_params=pltpu.CompilerParams(use_tc_tiling_on_sc=False,
                                             needs_layout_passes=False),
        out_type=jax.ShapeDtypeStruct((NB * _PW,), jnp.float32),
        scratch_types=[
            pltpu.VMEM((16, 8, 4, 32), jnp.int32),
            pltpu.VMEM((_PPW * _NI,), jnp.int32),
            pltpu.VMEM((_NI, D), jnp.float32),
            pltpu.VMEM((_PW,), jnp.float32),
            pltpu.SemaphoreType.DMA,
        ],
    )
    def gather_kernel(idx_hbm, table_hbm, out_hbm, staged_v, ordered_v,
                      rows_v, out_v, sem):
        wid = lax.axis_index("s") * _NC + lax.axis_index("c")
        bt = wid // 4
        bl0 = (wid % 4) * 32
        iot = jnp.arange(16, dtype=jnp.int32)

        for half in range(2):
            pltpu.sync_copy(
                idx_hbm.at[:, pl.ds(half * 8, 8), bt, :, pl.ds(bl0, 32)],
                staged_v)

            @plsc.parallel_loop(0, 128, unroll=4)
            def detr(g):
                l1 = g >> 3
                l2h = g & 7
                i0 = (l1 * 16 + half * 8 + l2h) * 4
                for o in range(4):
                    dbase = iot * 1024 + (i0 + o)
                    lo = staged_v[l1, l2h, o, pl.ds(0, 16)]
                    plsc.store_scatter(ordered_v, [dbase], lo)
                    hi = staged_v[l1, l2h, o, pl.ds(16, 16)]
                    plsc.store_scatter(ordered_v, [dbase + 16 * 1024], hi)

        cvec = (iot // 8) * 8192 + (iot % 8) * 128

        def plane(p, carry):
            row0 = p * _NI
            copies = []
            for j in range(_KB):
                copies.append(pltpu.async_copy(
                    table_hbm.at[ordered_v.at[pl.ds(row0 + j * _G, _G)]],
                    rows_v.at[pl.ds(j * _G, _G)],
                    sem,
                ))
            for c in copies:
                c.wait()

            @plsc.parallel_loop(0, _NI, unroll=8)
            def transp(i):
                base = (i >> 7) * 1024 + (i & 127)
                lo = rows_v[i, pl.ds(0, 16)]
                plsc.store_scatter(out_v, [cvec + base], lo)
                hi = rows_v[i, pl.ds(16, 16)]
                plsc.store_scatter(out_v, [cvec + (base + 16384)], hi)
            pltpu.sync_copy(out_v,
                            out_hbm.at[pl.ds((wid * _PPW + p) * _PW, _PW)])
            return carry

        lax.fori_loop(0, _PPW, plane, 0)

    return gather_kernel


def kernel(x, parameter):
    L1, L2, orbit_num = x.shape[-3], x.shape[-2], x.shape[-1]
    lead = x.shape[:-3]
    NI = L1 * L2 * orbit_num
    V, D = parameter.shape

    x_phys = (x.transpose(1, 2, 3, 0)
               .reshape(L1, L2, orbit_num, 8, 128)
               .transpose(0, 1, 3, 2, 4))

    out1d = _make_gather(V, D)(x_phys, parameter)
    out5 = out1d.reshape(lead[0], 4, 8, 8, 128)
    result = out5.transpose(0, 2, 4, 1, 3).reshape(lead + (NI, D))

    p1 = jnp.broadcast_to(jnp.arange(L1).reshape(L1, 1, 1), (L1, L2, orbit_num))
    p2 = jnp.broadcast_to(jnp.arange(L2).reshape(1, L2, 1), (L1, L2, orbit_num))
    p3 = jnp.broadcast_to(jnp.arange(orbit_num).reshape(1, 1, orbit_num),
                          (L1, L2, orbit_num))
    position = jnp.stack([p1, p2, p3]).reshape(3, NI)
    position = position.transpose(1, 0).astype(result.dtype)
    return (result, position)


# speedup vs baseline: 13.9298x; 2.4519x over previous
import functools

import jax
import jax.numpy as jnp
from jax import lax
from jax.experimental import pallas as pl
from jax.experimental.pallas import tpu as pltpu
from jax.experimental.pallas import tpu_sc as plsc

_NC = 2
_NS = 16
_NW = _NC * _NS
_G = 128
_KB = 8
_NI = _G * _KB
_PPW = 32
_OS = 1041


@functools.lru_cache(maxsize=None)
def _make_gather(V, D):
    NB = _NW * _PPW
    mesh = plsc.VectorSubcoreMesh(core_axis_name="c", subcore_axis_name="s")

    @functools.partial(
        pl.kernel,
        mesh=mesh,
        compiler_params=pltpu.CompilerParams(use_tc_tiling_on_sc=False,
                                             needs_layout_passes=False),
        out_type=jax.ShapeDtypeStruct((NB, 32, 8, 128), jnp.float32),
        scratch_types=[
            pltpu.VMEM((16, 8, 4, 32), jnp.int32),
            pltpu.VMEM((_PPW, _OS), jnp.int32),
            pltpu.VMEM((_NI, D), jnp.float32),
            pltpu.VMEM((32, 9, 137), jnp.float32),
            pltpu.SemaphoreType.DMA,
        ],
    )
    def gather_kernel(idx_hbm, table_hbm, out_hbm, staged_v, ordered_v,
                      rows_v, out_v, sem):
        wid = lax.axis_index("s") * _NC + lax.axis_index("c")
        bt = wid // 4
        bl0 = (wid % 4) * 32
        iot = jnp.arange(16, dtype=jnp.int32)

        row_lo = iot
        row_hi = iot + 16
        for half in range(2):
            pltpu.sync_copy(
                idx_hbm.at[:, pl.ds(half * 8, 8), bt, :, pl.ds(bl0, 32)],
                staged_v)

            @plsc.parallel_loop(0, 128, unroll=4)
            def detr(g):
                l1 = g >> 3
                l2h = g & 7
                i0 = (l1 * 16 + half * 8 + l2h) * 4
                for o in range(4):
                    col = jnp.full((16,), i0 + o, jnp.int32)
                    lo = staged_v[l1, l2h, o, pl.ds(0, 16)]
                    plsc.store_scatter(ordered_v, [row_lo, col], lo)
                    hi = staged_v[l1, l2h, o, pl.ds(16, 16)]
                    plsc.store_scatter(ordered_v, [row_hi, col], hi)

        cdt8 = (iot >> 3) * 8
        cdr = iot & 7

        def plane(p, carry):
            copies = []
            for j in range(_KB):
                copies.append(pltpu.async_copy(
                    table_hbm.at[ordered_v.at[p, pl.ds(j * _G, _G)]],
                    rows_v.at[pl.ds(j * _G, _G)],
                    sem,
                ))
            for c in copies:
                c.wait()

            @plsc.parallel_loop(0, _NI, unroll=8)
            def transp(i):
                it = i >> 7
                il = jnp.full((16,), i & 127, jnp.int32)
                dtit_lo = cdt8 + it
                lo = rows_v[i, pl.ds(0, 16)]
                plsc.store_scatter(out_v, [dtit_lo, cdr, il], lo)
                hi = rows_v[i, pl.ds(16, 16)]
                plsc.store_scatter(out_v, [dtit_lo + 16, cdr, il], hi)

            pltpu.sync_copy(out_v.at[:, pl.ds(0, 8), pl.ds(0, 128)],
                            out_hbm.at[wid * _PPW + p])
            return carry

        lax.fori_loop(0, _PPW, plane, 0)

    return gather_kernel


def kernel(x, parameter):
    L1, L2, orbit_num = x.shape[-3], x.shape[-2], x.shape[-1]
    lead = x.shape[:-3]
    NI = L1 * L2 * orbit_num
    V, D = parameter.shape

    x_phys = (x.transpose(1, 2, 3, 0)
               .reshape(L1, L2, orbit_num, 8, 128)
               .transpose(0, 1, 3, 2, 4))

    out4 = _make_gather(V, D)(x_phys, parameter)
    out5 = out4.reshape(lead[0], 4, 8, 8, 128)
    result = out5.transpose(0, 2, 4, 1, 3).reshape(lead + (NI, D))

    p1 = jnp.broadcast_to(jnp.arange(L1).reshape(L1, 1, 1), (L1, L2, orbit_num))
    p2 = jnp.broadcast_to(jnp.arange(L2).reshape(1, L2, 1), (L1, L2, orbit_num))
    p3 = jnp.broadcast_to(jnp.arange(orbit_num).reshape(1, 1, orbit_num),
                          (L1, L2, orbit_num))
    position = jnp.stack([p1, p2, p3]).reshape(3, NI)
    position = position.transpose(1, 0).astype(result.dtype)
    return (result, position)


# speedup vs baseline: 18.0385x; 1.2950x over previous
import functools

import jax
import jax.numpy as jnp
from jax import lax
from jax.experimental import pallas as pl
from jax.experimental.pallas import tpu as pltpu
from jax.experimental.pallas import tpu_sc as plsc

_NC = 2
_NS = 16
_NW = _NC * _NS
_G = 128
_KB = 8
_NI = _G * _KB
_PPW = 32
_OS = 1041


@functools.lru_cache(maxsize=None)
def _make_gather(V, D):
    NB = _NW * _PPW
    mesh = plsc.VectorSubcoreMesh(core_axis_name="c", subcore_axis_name="s")

    @functools.partial(
        pl.kernel,
        mesh=mesh,
        compiler_params=pltpu.CompilerParams(use_tc_tiling_on_sc=False,
                                             needs_layout_passes=False),
        out_type=jax.ShapeDtypeStruct((NB, 32, 8, 128), jnp.float32),
        scratch_types=[
            pltpu.VMEM((16, 8, 4, 32), jnp.int32),
            pltpu.VMEM((_PPW, _OS), jnp.int32),
            pltpu.VMEM((512, D), jnp.float32),
            pltpu.VMEM((512, D), jnp.float32),
            pltpu.VMEM((32, 9, 137), jnp.float32),
            pltpu.SemaphoreType.DMA,
            pltpu.SemaphoreType.DMA,
            pltpu.SemaphoreType.DMA,
        ],
    )
    def gather_kernel(idx_hbm, table_hbm, out_hbm, staged_v, ordered_v,
                      rows_a, rows_b, out_v, sem_a, sem_b, sem_w):
        wid = lax.axis_index("s") * _NC + lax.axis_index("c")
        bt = wid // 4
        bl0 = (wid % 4) * 32
        iot = jnp.arange(16, dtype=jnp.int32)

        row_lo = iot
        row_hi = iot + 16
        for half in range(2):
            pltpu.sync_copy(
                idx_hbm.at[:, pl.ds(half * 8, 8), bt, :, pl.ds(bl0, 32)],
                staged_v)

            @plsc.parallel_loop(0, 128, unroll=4)
            def detr(g):
                l1 = g >> 3
                l2h = g & 7
                i0 = (l1 * 16 + half * 8 + l2h) * 4
                for o in range(4):
                    col = jnp.full((16,), i0 + o, jnp.int32)
                    lo = staged_v[l1, l2h, o, pl.ds(0, 16)]
                    plsc.store_scatter(ordered_v, [row_lo, col], lo)
                    hi = staged_v[l1, l2h, o, pl.ds(16, 16)]
                    plsc.store_scatter(ordered_v, [row_hi, col], hi)

        cdt8 = (iot >> 3) * 8
        cdr = iot & 7

        def fire(p, ph, buf, sem):
            for j in range(4):
                pltpu.make_async_copy(
                    table_hbm.at[ordered_v.at[p, pl.ds(ph * 512 + j * _G, _G)]],
                    buf.at[pl.ds(j * _G, _G)],
                    sem,
                ).start()

        def drain(p, ph, buf, sem):
            for j in range(4):
                pltpu.make_async_copy(
                    table_hbm.at[ordered_v.at[p, pl.ds(ph * 512 + j * _G, _G)]],
                    buf.at[pl.ds(j * _G, _G)],
                    sem,
                ).wait()

        def transp_half(ph, buf):
            @plsc.parallel_loop(0, 512, unroll=8)
            def transp(r):
                i = ph * 512 + r
                it = i >> 7
                il = jnp.full((16,), i & 127, jnp.int32)
                dtit_lo = cdt8 + it
                lo = buf[r, pl.ds(0, 16)]
                plsc.store_scatter(out_v, [dtit_lo, cdr, il], lo)
                hi = buf[r, pl.ds(16, 16)]
                plsc.store_scatter(out_v, [dtit_lo + 16, cdr, il], hi)

        def write(p):
            c = pltpu.make_async_copy(
                out_v.at[:, pl.ds(0, 8), pl.ds(0, 128)],
                out_hbm.at[wid * _PPW + p],
                sem_w,
            )
            return c

        fire(0, 0, rows_a, sem_a)

        def pair(p, carry):
            fire(p, 1, rows_b, sem_b)
            drain(p, 0, rows_a, sem_a)

            @pl.when(p > 0)
            def _():
                write(p - 1).wait()

            transp_half(0, rows_a)

            @pl.when(p < _PPW - 1)
            def _():
                fire(p + 1, 0, rows_a, sem_a)

            drain(p, 1, rows_b, sem_b)
            transp_half(1, rows_b)
            write(p).start()
            return carry

        lax.fori_loop(0, _PPW, pair, 0)
        write(_PPW - 1).wait()

    return gather_kernel


def kernel(x, parameter):
    L1, L2, orbit_num = x.shape[-3], x.shape[-2], x.shape[-1]
    lead = x.shape[:-3]
    NI = L1 * L2 * orbit_num
    V, D = parameter.shape

    x_phys = (x.transpose(1, 2, 3, 0)
               .reshape(L1, L2, orbit_num, 8, 128)
               .transpose(0, 1, 3, 2, 4))

    out4 = _make_gather(V, D)(x_phys, parameter)
    out5 = out4.reshape(lead[0], 4, 8, 8, 128)
    result = out5.transpose(0, 2, 4, 1, 3).reshape(lead + (NI, D))

    p1 = jnp.broadcast_to(jnp.arange(L1).reshape(L1, 1, 1), (L1, L2, orbit_num))
    p2 = jnp.broadcast_to(jnp.arange(L2).reshape(1, L2, 1), (L1, L2, orbit_num))
    p3 = jnp.broadcast_to(jnp.arange(orbit_num).reshape(1, 1, orbit_num),
                          (L1, L2, orbit_num))
    position = jnp.stack([p1, p2, p3]).reshape(3, NI)
    position = position.transpose(1, 0).astype(result.dtype)
    return (result, position)
